# baseline (speedup 1.0000x reference)
import jax
import jax.numpy as jnp
from jax import lax
from jax.experimental import pallas as pl
from jax.experimental.pallas import tpu as pltpu
from jax.experimental.pallas import tpu_sc as plsc

V = 1_000_000
D = 64
B = 16_384
NC = 2
NS = 16
NW = NC * NS
L = 16

CB = 1_536
NBLK = V // CB
TAILC = V - NBLK * CB
BASE_NB = NBLK // NW
EXTRA = NBLK % NW
NSTEP = BASE_NB + 1
IDX_PIECE = 1_024
NPIECE = B // IDX_PIECE
CAP = B + 2 * L
PADSLOT = CAP - 1
C2CAP = 512
POSBITS = 14


def _body(xt_hbm, idx_hbm, td_hbm, out_hbm,
          idxv, comb, comb2, pb_a, pb_b, tp0, tp1, xb, xbt, st_sem, tg_sem):
    c = lax.axis_index("c")
    s = lax.axis_index("s")
    w = s * NC + c
    lane = lax.iota(jnp.int32, L)
    nb = BASE_NB + (w < EXTRA).astype(jnp.int32)
    start = BASE_NB * w + jnp.minimum(w, EXTRA)
    cs = start * CB
    ce = cs + nb * CB + jnp.where(w == NW - 1, TAILC, 0)

    def piece_body(p, cnt):
        pltpu.sync_copy(
            idx_hbm.at[pl.ds(pl.multiple_of(p * IDX_PIECE, 8), IDX_PIECE)],
            idxv)

        def scan_body(v, cnt):
            idx16 = idxv[pl.ds(pl.multiple_of(v * L, 8), L)]
            m = (idx16 >= cs) & (idx16 < ce)
            pcnt = plsc.all_reduce_population_count(m)[0]

            def with_matches(cnt):
                mi = m.astype(jnp.int32)
                off = plsc.cumsum(mi) - mi
                dst = jnp.where(m, cnt + off, PADSLOT)
                packed = ((idx16 - cs) << POSBITS) | \
                    (lane + (p * IDX_PIECE + v * L))
                plsc.store_scatter(comb, [dst], packed, mask=m)
                return cnt + pcnt

            return lax.cond(pcnt > 0, with_matches, lambda cnt: cnt, cnt)

        return lax.fori_loop(0, IDX_PIECE // L, scan_body, cnt)

    cnt = lax.fori_loop(0, NPIECE, piece_body, jnp.int32(0))
    nfb = (cnt + (L - 1)) // L

    def filt_round(r, lo_col, width):
        def filt_body(fb, c2):
            e16 = comb[pl.ds(pl.multiple_of(fb * L, 8), L)]
            col = e16 >> POSBITS
            m = ((fb * L + lane) < cnt) & (col >= lo_col) & \
                (col < lo_col + width)
            mi = m.astype(jnp.int32)
            off = plsc.cumsum(mi) - mi
            rank = c2 + off
            m2 = m & (rank >= r * C2CAP) & (rank < (r + 1) * C2CAP)
            dst = jnp.where(m2, rank - r * C2CAP, C2CAP + L - 1)
            plsc.store_scatter(comb2, [dst], e16, mask=m2)
            return c2 + plsc.all_reduce_population_count(m)[0]

        return lax.fori_loop(0, nfb, filt_body, jnp.int32(0))

    def issue_prefetch():
        e0 = comb2[pl.ds(0, L)]
        e1 = comb2[pl.ds(L, L)]
        pb_a[...] = e0 & ((1 << POSBITS) - 1)
        pb_b[...] = e1 & ((1 << POSBITS) - 1)
        pltpu.async_copy(td_hbm.at[pb_a], tp0, tg_sem)
        pltpu.async_copy(td_hbm.at[pb_b], tp1, tg_sem)

    def wait_prefetch():
        pltpu.make_async_copy(td_hbm.at[pb_a], tp0, tg_sem).wait()
        pltpu.make_async_copy(td_hbm.at[pb_b], tp1, tg_sem).wait()

    def apply16(buf, e16, valid, lo_col, tp):
        colloc = (e16 >> POSBITS) - lo_col
        for q in range(D):
            qs = jnp.full((L,), q, jnp.int32)
            vals = plsc.load_gather(tp, [lane, qs])
            plsc.addupdate_scatter(buf, [qs, colloc], vals, mask=valid)

    def sync_batch(buf, b, rcnt, lo_col):
        e16 = comb2[pl.ds(pl.multiple_of(b * L, 8), L)]
        valid = (b * L + lane) < rcnt
        pb_a[...] = jnp.where(valid, e16 & ((1 << POSBITS) - 1), 0)
        pltpu.sync_copy(td_hbm.at[pb_a], tp0)
        apply16(buf, e16, valid, lo_col, tp0)

    def apply_block(buf, lo_col, total2, prefetched):
        rcnt = jnp.minimum(total2, C2CAP)
        if prefetched:
            e0 = comb2[pl.ds(0, L)]
            apply16(buf, e0, lane < rcnt, lo_col, tp0)

            @pl.when(rcnt > L)
            def _():
                e1 = comb2[pl.ds(L, L)]
                apply16(buf, e1, (L + lane) < rcnt, lo_col, tp1)

            lax.fori_loop(2, (rcnt + (L - 1)) // L,
                          lambda b, cc: (sync_batch(buf, b, rcnt, lo_col), cc)[1],
                          jnp.int32(0))
        else:
            lax.fori_loop(0, (rcnt + (L - 1)) // L,
                          lambda b, cc: (sync_batch(buf, b, rcnt, lo_col), cc)[1],
                          jnp.int32(0))

        def round_body(r, carry):
            filt_round(r, lo_col, CB)
            rc = jnp.minimum(total2 - r * C2CAP, C2CAP)
            lax.fori_loop(0, (rc + (L - 1)) // L,
                          lambda b, cc: (sync_batch(buf, b, rc, lo_col), cc)[1],
                          jnp.int32(0))
            return carry

        lax.fori_loop(1, (total2 + (C2CAP - 1)) // C2CAP, round_body,
                      jnp.int32(0))

    t2_0 = filt_round(0, jnp.int32(0), CB)
    issue_prefetch()

    def step_body(j, total2):
        @pl.when(j < nb)
        def _():
            @pl.when(j >= 1)
            def _():
                pltpu.make_async_copy(
                    xb, out_hbm.at[:, pl.ds(0, CB)], st_sem).wait()
            pltpu.sync_copy(
                xt_hbm.at[:, pl.ds(pl.multiple_of((start + j) * CB, 128),
                                   CB)], xb)
            wait_prefetch()
            apply_block(xb, j * CB, total2, True)
            pltpu.async_copy(
                xb, out_hbm.at[:, pl.ds(pl.multiple_of((start + j) * CB, 128),
                                        CB)], st_sem)

        nxt = lax.cond(j + 1 < nb,
                       lambda: filt_round(0, (j + 1) * CB, CB),
                       lambda: total2)

        @pl.when(j + 1 < nb)
        def _():
            issue_prefetch()
        return nxt

    lax.fori_loop(0, NSTEP, step_body, t2_0)
    pltpu.make_async_copy(xb, out_hbm.at[:, pl.ds(0, CB)], st_sem).wait()

    @pl.when(w == NW - 1)
    def _():
        o = NBLK * CB
        pltpu.sync_copy(xt_hbm.at[:, pl.ds(o, TAILC)], xbt)
        t2t = filt_round(0, nb * CB, TAILC)
        rct = jnp.minimum(t2t, C2CAP)
        lax.fori_loop(0, (rct + (L - 1)) // L,
                      lambda b, cc: (sync_batch(xbt, b, rct, nb * CB), cc)[1],
                      jnp.int32(0))

        def round_body(r, carry):
            filt_round(r, nb * CB, TAILC)
            rc = jnp.minimum(t2t - r * C2CAP, C2CAP)
            lax.fori_loop(0, (rc + (L - 1)) // L,
                          lambda b, cc: (sync_batch(xbt, b, rc, nb * CB),
                                         cc)[1], jnp.int32(0))
            return carry

        lax.fori_loop(1, (t2t + (C2CAP - 1)) // C2CAP, round_body,
                      jnp.int32(0))
        pltpu.sync_copy(xbt, out_hbm.at[:, pl.ds(o, TAILC)])


@jax.jit
def _index_add(xt, idx32, tdup):
    mesh = plsc.VectorSubcoreMesh(core_axis_name="c", subcore_axis_name="s")
    f = pl.kernel(
        _body,
        out_type=jax.ShapeDtypeStruct((D, V), jnp.float32),
        mesh=mesh,
        scratch_types=[
            pltpu.VMEM((IDX_PIECE,), jnp.int32),
            pltpu.VMEM((CAP,), jnp.int32),
            pltpu.VMEM((C2CAP + L,), jnp.int32),
            pltpu.VMEM((L,), jnp.int32),
            pltpu.VMEM((L,), jnp.int32),
            pltpu.VMEM((L, 2 * D), jnp.float32),
            pltpu.VMEM((L, 2 * D), jnp.float32),
            pltpu.VMEM((D, CB), jnp.float32),
            pltpu.VMEM((D, TAILC), jnp.float32),
            pltpu.SemaphoreType.DMA,
            pltpu.SemaphoreType.DMA,
        ],
        compiler_params=pltpu.CompilerParams(needs_layout_passes=False),
    )
    return f(xt, idx32, tdup)


def kernel(x, dim, index, t):
    idx32 = (index + dim).astype(jnp.int32)
    tdup = jnp.concatenate([t, t], axis=1)
    outT = _index_add(x.T, idx32, tdup)
    return outT.T


# speedup vs baseline: 1.2780x; 1.2780x over previous
import jax
import jax.numpy as jnp
from jax import lax
from jax.experimental import pallas as pl
from jax.experimental.pallas import tpu as pltpu
from jax.experimental.pallas import tpu_sc as plsc

V = 1_000_000
D = 64
B = 16_384
NC = 2
NS = 16
NW = NC * NS
L = 16

CB = 768
NBLK = V // CB
TAILC = V - NBLK * CB
BASE_NB = NBLK // NW
EXTRA = NBLK % NW
NSTEP = BASE_NB + 1
NPAIR = (NSTEP + 1) // 2
IDX_PIECE = 1_024
NPIECE = B // IDX_PIECE
CAP = B + 2 * L
PADSLOT = CAP - 1
C2CAP = 512
POSBITS = 14


def _body(xt_hbm, idx_hbm, td_hbm, out_hbm,
          idxv, comb, comb2, pb_a, pb_b, tp0, tp1, xbA, xbB, xbt,
          ldA, ldB, stA, stB, tg_sem):
    c = lax.axis_index("c")
    s = lax.axis_index("s")
    w = s * NC + c
    lane = lax.iota(jnp.int32, L)
    nb = BASE_NB + (w < EXTRA).astype(jnp.int32)
    start = BASE_NB * w + jnp.minimum(w, EXTRA)
    cs = start * CB
    ce = cs + nb * CB + jnp.where(w == NW - 1, TAILC, 0)

    def blk_off(j):
        return pl.multiple_of((start + j) * CB, 128)

    def piece_body(p, cnt):
        pltpu.sync_copy(
            idx_hbm.at[pl.ds(pl.multiple_of(p * IDX_PIECE, 8), IDX_PIECE)],
            idxv)

        def scan_body(v, cnt):
            idx16 = idxv[pl.ds(pl.multiple_of(v * L, 8), L)]
            m = (idx16 >= cs) & (idx16 < ce)
            pcnt = plsc.all_reduce_population_count(m)[0]

            def with_matches(cnt):
                mi = m.astype(jnp.int32)
                off = plsc.cumsum(mi) - mi
                dst = jnp.where(m, cnt + off, PADSLOT)
                packed = ((idx16 - cs) << POSBITS) | \
                    (lane + (p * IDX_PIECE + v * L))
                plsc.store_scatter(comb, [dst], packed, mask=m)
                return cnt + pcnt

            return lax.cond(pcnt > 0, with_matches, lambda cnt: cnt, cnt)

        return lax.fori_loop(0, IDX_PIECE // L, scan_body, cnt)

    cnt = lax.fori_loop(0, NPIECE, piece_body, jnp.int32(0))
    nfb = (cnt + (L - 1)) // L

    def filt_round(r, lo_col, width):
        def filt_body(fb, c2):
            e16 = comb[pl.ds(pl.multiple_of(fb * L, 8), L)]
            col = e16 >> POSBITS
            m = ((fb * L + lane) < cnt) & (col >= lo_col) & \
                (col < lo_col + width)
            mi = m.astype(jnp.int32)
            off = plsc.cumsum(mi) - mi
            rank = c2 + off
            m2 = m & (rank >= r * C2CAP) & (rank < (r + 1) * C2CAP)
            dst = jnp.where(m2, rank - r * C2CAP, C2CAP + L - 1)
            plsc.store_scatter(comb2, [dst], e16, mask=m2)
            return c2 + plsc.all_reduce_population_count(m)[0]

        return lax.fori_loop(0, nfb, filt_body, jnp.int32(0))

    def issue_prefetch():
        e0 = comb2[pl.ds(0, L)]
        e1 = comb2[pl.ds(L, L)]
        pb_a[...] = e0 & ((1 << POSBITS) - 1)
        pb_b[...] = e1 & ((1 << POSBITS) - 1)
        pltpu.async_copy(td_hbm.at[pb_a], tp0, tg_sem)
        pltpu.async_copy(td_hbm.at[pb_b], tp1, tg_sem)

    def wait_prefetch():
        pltpu.make_async_copy(td_hbm.at[pb_a], tp0, tg_sem).wait()
        pltpu.make_async_copy(td_hbm.at[pb_b], tp1, tg_sem).wait()

    def apply16(buf, e16, valid, lo_col, tp):
        colloc = (e16 >> POSBITS) - lo_col
        for q in range(D):
            qs = jnp.full((L,), q, jnp.int32)
            vals = plsc.load_gather(tp, [lane, qs])
            plsc.addupdate_scatter(buf, [qs, colloc], vals, mask=valid)

    def sync_batch(buf, b, rcnt, lo_col):
        e16 = comb2[pl.ds(pl.multiple_of(b * L, 8), L)]
        valid = (b * L + lane) < rcnt
        pb_a[...] = jnp.where(valid, e16 & ((1 << POSBITS) - 1), 0)
        pltpu.sync_copy(td_hbm.at[pb_a], tp0)
        apply16(buf, e16, valid, lo_col, tp0)

    def apply_block(buf, lo_col, total2):
        rcnt = jnp.minimum(total2, C2CAP)
        e0 = comb2[pl.ds(0, L)]
        apply16(buf, e0, lane < rcnt, lo_col, tp0)

        @pl.when(rcnt > L)
        def _():
            e1 = comb2[pl.ds(L, L)]
            apply16(buf, e1, (L + lane) < rcnt, lo_col, tp1)

        lax.fori_loop(2, (rcnt + (L - 1)) // L,
                      lambda b, cc: (sync_batch(buf, b, rcnt, lo_col), cc)[1],
                      jnp.int32(0))

        def round_body(r, carry):
            filt_round(r, lo_col, CB)
            rc = jnp.minimum(total2 - r * C2CAP, C2CAP)
            lax.fori_loop(0, (rc + (L - 1)) // L,
                          lambda b, cc: (sync_batch(buf, b, rc, lo_col),
                                         cc)[1], jnp.int32(0))
            return carry

        lax.fori_loop(1, (total2 + (C2CAP - 1)) // C2CAP, round_body,
                      jnp.int32(0))

    def half_step(j, bufX, ld_X, st_X, bufY, ld_Y, st_Y, total2):
        @pl.when(j < nb)
        def _():
            pltpu.make_async_copy(
                xt_hbm.at[:, pl.ds(blk_off(j), CB)], bufX, ld_X).wait()

            @pl.when(j + 1 < nb)
            def _():
                @pl.when(j >= 1)
                def _():
                    pltpu.make_async_copy(
                        bufY, out_hbm.at[:, pl.ds(0, CB)], st_Y).wait()
                pltpu.async_copy(
                    xt_hbm.at[:, pl.ds(blk_off(j + 1), CB)], bufY, ld_Y)

            wait_prefetch()
            apply_block(bufX, j * CB, total2)
            pltpu.async_copy(
                bufX, out_hbm.at[:, pl.ds(blk_off(j), CB)], st_X)

        nxt = lax.cond(j + 1 < nb,
                       lambda: filt_round(0, (j + 1) * CB, CB),
                       lambda: total2)

        @pl.when(j + 1 < nb)
        def _():
            issue_prefetch()
        return nxt

    t2_0 = filt_round(0, jnp.int32(0), CB)
    issue_prefetch()
    pltpu.async_copy(xt_hbm.at[:, pl.ds(blk_off(0), CB)], xbA, ldA)

    def pair_body(g, total2):
        t2a = half_step(2 * g, xbA, ldA, stA, xbB, ldB, stB, total2)
        t2b = half_step(2 * g + 1, xbB, ldB, stB, xbA, ldA, stA, t2a)
        return t2b

    lax.fori_loop(0, NPAIR, pair_body, t2_0)
    pltpu.make_async_copy(xbA, out_hbm.at[:, pl.ds(0, CB)], stA).wait()
    pltpu.make_async_copy(xbB, out_hbm.at[:, pl.ds(0, CB)], stB).wait()

    @pl.when(w == NW - 1)
    def _():
        o = NBLK * CB
        pltpu.sync_copy(xt_hbm.at[:, pl.ds(o, TAILC)], xbt)
        t2t = filt_round(0, nb * CB, TAILC)
        rct = jnp.minimum(t2t, C2CAP)
        lax.fori_loop(0, (rct + (L - 1)) // L,
                      lambda b, cc: (sync_batch(xbt, b, rct, nb * CB), cc)[1],
                      jnp.int32(0))

        def round_body(r, carry):
            filt_round(r, nb * CB, TAILC)
            rc = jnp.minimum(t2t - r * C2CAP, C2CAP)
            lax.fori_loop(0, (rc + (L - 1)) // L,
                          lambda b, cc: (sync_batch(xbt, b, rc, nb * CB),
                                         cc)[1], jnp.int32(0))
            return carry

        lax.fori_loop(1, (t2t + (C2CAP - 1)) // C2CAP, round_body,
                      jnp.int32(0))
        pltpu.sync_copy(xbt, out_hbm.at[:, pl.ds(o, TAILC)])


@jax.jit
def _index_add(xt, idx32, tdup):
    mesh = plsc.VectorSubcoreMesh(core_axis_name="c", subcore_axis_name="s")
    f = pl.kernel(
        _body,
        out_type=jax.ShapeDtypeStruct((D, V), jnp.float32),
        mesh=mesh,
        scratch_types=[
            pltpu.VMEM((IDX_PIECE,), jnp.int32),
            pltpu.VMEM((CAP,), jnp.int32),
            pltpu.VMEM((C2CAP + L,), jnp.int32),
            pltpu.VMEM((L,), jnp.int32),
            pltpu.VMEM((L,), jnp.int32),
            pltpu.VMEM((L, 2 * D), jnp.float32),
            pltpu.VMEM((L, 2 * D), jnp.float32),
            pltpu.VMEM((D, CB), jnp.float32),
            pltpu.VMEM((D, CB), jnp.float32),
            pltpu.VMEM((D, TAILC), jnp.float32),
            pltpu.SemaphoreType.DMA,
            pltpu.SemaphoreType.DMA,
            pltpu.SemaphoreType.DMA,
            pltpu.SemaphoreType.DMA,
            pltpu.SemaphoreType.DMA,
        ],
        compiler_params=pltpu.CompilerParams(needs_layout_passes=False),
    )
    return f(xt, idx32, tdup)


def kernel(x, dim, index, t):
    idx32 = (index + dim).astype(jnp.int32)
    tdup = jnp.concatenate([t, t], axis=1)
    outT = _index_add(x.T, idx32, tdup)
    return outT.T
